# baseline (speedup 1.0000x reference)
import functools

import jax
import jax.numpy as jnp
from jax import lax
from jax.experimental import pallas as pl
from jax.experimental.pallas import tpu as pltpu
from jax.experimental.pallas import tpu_sc as plsc

_NC = 2
_NS = 16
_NW = _NC * _NS
_CHUNK = 2
_NBUF = 4


def _sc_body(table_hbm, idx_hbm, out_hbm, idx_v, bufs, sem_g, sem_s,
             *, nchunks_per_w):
    wid = lax.axis_index("s") * _NC + lax.axis_index("c")
    base = wid * nchunks_per_w
    pltpu.sync_copy(idx_hbm.at[pl.ds(base, nchunks_per_w)], idx_v)

    def gather_start(j, b):
        pltpu.async_copy(table_hbm.at[idx_v.at[j]], bufs.at[b], sem_g.at[b])

    def gather_wait(b):
        pltpu.make_async_copy(
            table_hbm.at[idx_v.at[0]], bufs.at[b], sem_g.at[b]).wait()

    def scatter_start(j, b):
        pltpu.async_copy(
            bufs.at[b], out_hbm.at[pl.ds((base + j) * _CHUNK, _CHUNK)],
            sem_s.at[b])

    def scatter_wait(b):
        pltpu.make_async_copy(
            bufs.at[b], out_hbm.at[pl.ds(0, _CHUNK)], sem_s.at[b]).wait()

    for b in range(_NBUF - 1):
        gather_start(b, b)

    @pl.loop(0, nchunks_per_w, step=_NBUF)
    def _pipeline(jj):
        for b in range(_NBUF):
            j = jj + b
            bp = (b + _NBUF - 1) % _NBUF
            gather_wait(b)
            scatter_start(j, b)

            @pl.when(j + _NBUF - 1 < nchunks_per_w)
            def _refill():
                @pl.when(j >= 1)
                def _free_buf():
                    scatter_wait(bp)

                gather_start(j + _NBUF - 1, bp)

    for b in range(_NBUF):
        scatter_wait(b)


def kernel(prefix, table):
    bsz, seq = prefix.shape
    n = bsz * seq
    vocab, width = table.shape
    nchunks_per_w = n // (_NW * _CHUNK)

    idx2 = prefix.reshape(n // _CHUNK, _CHUNK).astype(jnp.int32)
    mesh = plsc.VectorSubcoreMesh(core_axis_name="c", subcore_axis_name="s")
    body = functools.partial(_sc_body, nchunks_per_w=nchunks_per_w)
    k = pl.kernel(
        body,
        out_type=jax.ShapeDtypeStruct((n, width), table.dtype),
        mesh=mesh,
        scratch_types=[
            pltpu.VMEM((nchunks_per_w, _CHUNK), jnp.int32),
            pltpu.VMEM((_NBUF, _CHUNK, width), table.dtype),
            pltpu.SemaphoreType.DMA((_NBUF,)),
            pltpu.SemaphoreType.DMA((_NBUF,)),
        ],
    )
    out = k(table, idx2)
    return out.reshape(bsz, seq, width)


# speedup vs baseline: 1.2890x; 1.2890x over previous
import functools

import jax
import jax.numpy as jnp
from jax import lax
from jax.experimental import pallas as pl
from jax.experimental.pallas import tpu as pltpu
from jax.experimental.pallas import tpu_sc as plsc

_NC = 2
_NS = 16
_NW = _NC * _NS
_POS_CHUNK = 1024


def _sc_body(table_hbm, idx_hbm, out_hbm, my_rows, idx_v, sem_out,
             *, n, vocab):
    wid = lax.axis_index("s") * _NC + lax.axis_index("c")
    rpt = vocab // _NW
    lo = wid * rpt

    pltpu.sync_copy(table_hbm.at[pl.ds(lo, rpt)], my_rows)

    def scan_chunk(c, cnt):
        pltpu.sync_copy(idx_hbm.at[pl.ds(c * _POS_CHUNK, _POS_CHUNK)], idx_v)

        def scan_vec(v, cnt_in):
            off = pl.multiple_of(v * 16, 16)
            lvec = idx_v[pl.ds(off, 16)] - lo
            hits = jnp.logical_and(lvec >= 0, lvec < rpt).astype(jnp.int32)
            nhit = jnp.sum(hits, axis=0)

            @pl.when(nhit > 0)
            def _lanes():
                for lane in range(16):
                    l = lvec[lane]

                    @pl.when(jnp.logical_and(l >= 0, l < rpt))
                    def _push():
                        pltpu.async_copy(
                            my_rows.at[l],
                            out_hbm.at[c * _POS_CHUNK + v * 16 + lane],
                            sem_out)

            return cnt_in + nhit

        return pl.loop(0, _POS_CHUNK // 16, init_carry=cnt)(scan_vec)

    total = pl.loop(0, n // _POS_CHUNK, init_carry=jnp.int32(0))(scan_chunk)

    def drain(_i, carry):
        pltpu.make_async_copy(my_rows.at[0], out_hbm.at[0], sem_out).wait()
        return carry

    pl.loop(0, total, init_carry=jnp.int32(0))(drain)


def kernel(prefix, table):
    bsz, seq = prefix.shape
    n = bsz * seq
    vocab, width = table.shape

    idx = prefix.reshape(n).astype(jnp.int32)
    mesh = plsc.VectorSubcoreMesh(core_axis_name="c", subcore_axis_name="s")
    body = functools.partial(_sc_body, n=n, vocab=vocab)
    k = pl.kernel(
        body,
        out_type=jax.ShapeDtypeStruct((n, width), table.dtype),
        mesh=mesh,
        compiler_params=pltpu.CompilerParams(needs_layout_passes=False),
        scratch_types=[
            pltpu.VMEM((vocab // _NW, width), table.dtype),
            pltpu.VMEM((_POS_CHUNK,), jnp.int32),
            pltpu.SemaphoreType.DMA,
        ],
    )
    out = k(table, idx)
    return out.reshape(bsz, seq, width)
